# baseline (speedup 1.0000x reference)
import functools

import jax
import jax.numpy as jnp
from jax import lax
from jax.experimental import pallas as pl
from jax.experimental.pallas import tpu as pltpu
from jax.experimental.pallas import tpu_sc as plsc

_C = 512
_BINS = 10
_TBLK = 1024
_CH = 8
_NCH = _C // _CH

_NC = 2
_NS = 16
_NW = _NC * _NS
_L = 16


def _dense_body(x_ref, lbl_ref, raw_ref, bin_ref):
    lbl = lbl_ref[0]

    def mx_step(k, acc):
        return jnp.maximum(acc, x_ref[0, pl.ds(k * _CH, _CH), :])

    m8 = jax.lax.fori_loop(1, _NCH, mx_step, x_ref[0, 0:_CH, :])
    m = jnp.max(m8, axis=0, keepdims=True)

    iota8 = jax.lax.broadcasted_iota(jnp.int32, (_CH, _TBLK), 0)
    z8 = jnp.zeros((_CH, _TBLK), jnp.float32)

    def step(k, carry):
        s8, xl8 = carry
        x = x_ref[0, pl.ds(k * _CH, _CH), :]
        s8 = s8 + jnp.exp(x - m)
        xl8 = xl8 + jnp.where(iota8 == (lbl - k * _CH), x, 0.0)
        return s8, xl8

    s8, xl8 = jax.lax.fori_loop(0, _NCH, step, (z8, z8))
    s = jnp.sum(s8, axis=0, keepdims=True)
    xl = jnp.sum(xl8, axis=0, keepdims=True)
    lse = m + jnp.log(s)

    p = jnp.exp(xl - lse)
    gd = jnp.abs(p - 1.0)
    raw_ref[0, 0] = lse - xl
    bin_ref[0, 0] = jnp.clip(jnp.floor(gd * _BINS).astype(jnp.int32), 0, _BINS - 1)


def _sc_body(raw_hbm, lbl_hbm, bin_hbm, cls_hbm, gd_hbm, out_hbm,
             raw_v, lbl_v, bin_v, cls_v, gd_v, acc_v, pw):
    wid = lax.axis_index("s") * _NC + lax.axis_index("c")
    base = wid * pw
    pltpu.sync_copy(raw_hbm.at[pl.ds(base, pw)], raw_v)
    pltpu.sync_copy(lbl_hbm.at[pl.ds(base, pw)], lbl_v)
    pltpu.sync_copy(bin_hbm.at[pl.ds(base, pw)], bin_v)
    pltpu.sync_copy(cls_hbm, cls_v)
    pltpu.sync_copy(gd_hbm, gd_v)

    def step(i, acc):
        sl = pl.ds(i * _L, _L)
        cw = plsc.load_gather(cls_v, [lbl_v[sl]])
        gw = plsc.load_gather(gd_v, [bin_v[sl]])
        x = cw * gw
        ii = 0x5F3759DF - jnp.right_shift(plsc.bitcast(x, jnp.int32), 1)
        y = plsc.bitcast(ii, jnp.float32)
        y = y * (1.5 - 0.5 * x * y * y)
        y = y * (1.5 - 0.5 * x * y * y)
        y = y * (1.5 - 0.5 * x * y * y)
        inv = jnp.minimum(y, 1e10)
        return acc + raw_v[sl] * inv

    acc = jax.lax.fori_loop(0, pw // _L, step, jnp.zeros((_L,), jnp.float32))
    acc_v[...] = acc
    pltpu.sync_copy(acc_v, out_hbm.at[wid])


def kernel(pred_logits, target_label, GD_ema, class_ema):
    B, C, T = pred_logits.shape
    nT = T // _TBLK
    ntok = B * T
    pw = ntok // _NW

    lbl3 = target_label.reshape(B, 1, T)
    raw, bins = pl.pallas_call(
        _dense_body,
        grid=(B, nT),
        in_specs=[
            pl.BlockSpec((1, C, _TBLK), lambda b, t: (b, 0, t)),
            pl.BlockSpec((1, 1, _TBLK), lambda b, t: (b, 0, t)),
        ],
        out_specs=[
            pl.BlockSpec((1, 1, 1, _TBLK), lambda b, t: (b, t, 0, 0)),
            pl.BlockSpec((1, 1, 1, _TBLK), lambda b, t: (b, t, 0, 0)),
        ],
        out_shape=[
            jax.ShapeDtypeStruct((B, nT, 1, _TBLK), jnp.float32),
            jax.ShapeDtypeStruct((B, nT, 1, _TBLK), jnp.int32),
        ],
        compiler_params=pltpu.CompilerParams(
            dimension_semantics=("parallel", "parallel"),
        ),
    )(pred_logits, lbl3)

    gd16 = jnp.pad(GD_ema, (0, _L - _BINS))
    mesh = plsc.VectorSubcoreMesh(core_axis_name="c", subcore_axis_name="s")
    sc = pl.kernel(
        functools.partial(_sc_body, pw=pw),
        out_type=jax.ShapeDtypeStruct((_NW, _L), jnp.float32),
        mesh=mesh,
        scratch_types=[
            pltpu.VMEM((pw,), jnp.float32),
            pltpu.VMEM((pw,), jnp.int32),
            pltpu.VMEM((pw,), jnp.int32),
            pltpu.VMEM((_C,), jnp.float32),
            pltpu.VMEM((_L,), jnp.float32),
            pltpu.VMEM((_L,), jnp.float32),
        ],
        compiler_params=pltpu.CompilerParams(needs_layout_passes=False),
    )
    parts = sc(raw.reshape(ntok), target_label.reshape(ntok), bins.reshape(ntok),
               class_ema, gd16)
    return jnp.sum(parts) / ntok


# speedup vs baseline: 1.9329x; 1.9329x over previous
import jax
import jax.numpy as jnp
from jax.experimental import pallas as pl
from jax.experimental.pallas import tpu as pltpu

_C = 512
_BINS = 10
_TBLK = 2048
_BB = 2
_CH = 8
_NCH = _C // _CH


def _ghm_body(x_ref, lbl_ref, gd_ref, cls_ref, out_ref):
    total = jnp.zeros((1, 1), jnp.float32)
    for s in range(_BB):
        lbl = lbl_ref[s]

        m8 = x_ref[s, 0:_CH, :]
        for k in range(1, _NCH):
            m8 = jnp.maximum(m8, x_ref[s, k * _CH:(k + 1) * _CH, :])
        m = jnp.max(m8, axis=0, keepdims=True)
        mb = jnp.broadcast_to(m, (_CH, _TBLK))
        iota8 = jax.lax.broadcasted_iota(jnp.int32, (_CH, _TBLK), 0)
        lblb = jnp.broadcast_to(lbl, (_CH, _TBLK)) - iota8
        s8 = jnp.zeros((_CH, _TBLK), jnp.float32)
        xl8 = jnp.zeros((_CH, _TBLK), jnp.float32)
        cw8 = jnp.zeros((_CH, _TBLK), jnp.float32)
        for k in range(_NCH):
            x = x_ref[s, k * _CH:(k + 1) * _CH, :]
            hit = lblb == k * _CH
            s8 = s8 + jnp.exp(x - mb)
            xl8 = xl8 + jnp.where(hit, x, 0.0)
            cw8 = cw8 + jnp.where(hit, cls_ref[k * _CH:(k + 1) * _CH, :], 0.0)
        ssum = jnp.sum(s8, axis=0, keepdims=True)
        xl = jnp.sum(xl8, axis=0, keepdims=True)
        cw = jnp.sum(cw8, axis=0, keepdims=True)
        lse = m + jnp.log(ssum)

        p = jnp.exp(xl - lse)
        gd = jnp.abs(p - 1.0)
        idx = jnp.clip(jnp.floor(gd * _BINS).astype(jnp.int32), 0, _BINS - 1)
        gw = jnp.zeros_like(lse)
        for k in range(_BINS):
            gw = jnp.where(idx == k, gd_ref[0, k], gw)

        w = jnp.clip(jnp.sqrt(cw * gw), 1e-10, None)
        total = total + jnp.sum((lse - xl) / w, axis=1, keepdims=True)
    out_ref[0, 0] = total


def kernel(pred_logits, target_label, GD_ema, class_ema):
    B, C, T = pred_logits.shape
    nB = B // _BB
    nT = T // _TBLK
    lbl3 = target_label.reshape(B, 1, T)
    gd2 = GD_ema.reshape(1, _BINS)
    cls2 = class_ema.reshape(C, 1)
    parts = pl.pallas_call(
        _ghm_body,
        grid=(nB, nT),
        in_specs=[
            pl.BlockSpec((_BB, C, _TBLK), lambda b, t: (b, 0, t)),
            pl.BlockSpec((_BB, 1, _TBLK), lambda b, t: (b, 0, t)),
            pl.BlockSpec(memory_space=pltpu.SMEM),
            pl.BlockSpec((C, 1), lambda b, t: (0, 0)),
        ],
        out_specs=pl.BlockSpec((1, 1, 1, 1), lambda b, t: (b, t, 0, 0)),
        out_shape=jax.ShapeDtypeStruct((nB, nT, 1, 1), jnp.float32),
        compiler_params=pltpu.CompilerParams(
            dimension_semantics=("parallel", "parallel"),
        ),
    )(pred_logits, lbl3, gd2, cls2)
    return jnp.sum(parts) / (B * T)
